# baseline (speedup 1.0000x reference)
import functools

import jax
import jax.numpy as jnp
from jax import lax
from jax.experimental import pallas as pl
from jax.experimental.pallas import tpu as pltpu
from jax.experimental.pallas import tpu_sc as plsc

N_ATOMS = 10000
N_BONDS = 320000
MAX_NB = 32
HIDDEN = 128
DEPTH = 3

NC = 2
NS = 16
NW = NC * NS

ATOMS_PER_W = 320
A_PAD = NW * ATOMS_PER_W
SEG_ATOMS_PER_CHUNK = 4
SEG_CHUNKS = ATOMS_PER_W // SEG_ATOMS_PER_CHUNK

BONDS_PER_W = N_BONDS // NW
MSG_CHUNK = 80
MSG_CHUNKS = BONDS_PER_W // MSG_CHUNK

_mesh = plsc.VectorSubcoreMesh(core_axis_name="c", subcore_axis_name="s")


def _wid():
    return lax.axis_index("s") * NC + lax.axis_index("c")


@functools.partial(
    pl.kernel,
    out_type=jax.ShapeDtypeStruct((A_PAD, HIDDEN), jnp.float32),
    mesh=_mesh,
    scratch_types=[
        pltpu.VMEM((128,), jnp.int32),
        pltpu.VMEM((128, HIDDEN), jnp.float32),
        pltpu.VMEM((SEG_ATOMS_PER_CHUNK, HIDDEN), jnp.float32),
        pltpu.SemaphoreType.DMA,
    ],
)
def _seg_sum(h_hbm, idx_hbm, out_hbm, idx_v, rows_v, acc_v, sem):
    wid = _wid()

    def chunk(c, carry):
        ibase = wid * (ATOMS_PER_W * MAX_NB) + c * 128
        pltpu.sync_copy(idx_hbm.at[pl.ds(ibase, 128)], idx_v)
        pltpu.async_copy(h_hbm.at[idx_v], rows_v, sem).wait()
        for a in range(SEG_ATOMS_PER_CHUNK):
            def red(j, accs):
                return tuple(
                    accs[g] + rows_v[a * MAX_NB + j, pl.ds(g * 16, 16)]
                    for g in range(8)
                )
            accs = tuple(jnp.zeros((16,), jnp.float32) for _ in range(8))
            accs = lax.fori_loop(0, MAX_NB, red, accs)
            for g in range(8):
                acc_v[a, pl.ds(g * 16, 16)] = accs[g]
        abase = wid * ATOMS_PER_W + c * SEG_ATOMS_PER_CHUNK
        pltpu.sync_copy(acc_v, out_hbm.at[pl.ds(abase, SEG_ATOMS_PER_CHUNK)])
        return carry

    lax.fori_loop(0, SEG_CHUNKS, chunk, 0)


@functools.partial(
    pl.kernel,
    out_type=jax.ShapeDtypeStruct((N_BONDS, HIDDEN), jnp.float32),
    mesh=_mesh,
    scratch_types=[
        pltpu.VMEM((MSG_CHUNK,), jnp.int32),
        pltpu.VMEM((MSG_CHUNK,), jnp.int32),
        pltpu.VMEM((MSG_CHUNK, HIDDEN), jnp.float32),
        pltpu.VMEM((MSG_CHUNK, HIDDEN), jnp.float32),
        pltpu.VMEM((MSG_CHUNK, HIDDEN), jnp.float32),
        pltpu.VMEM((MSG_CHUNK, HIDDEN), jnp.float32),
        pltpu.SemaphoreType.DMA,
        pltpu.SemaphoreType.DMA,
    ],
)
def _msg_step(inp_hbm, h_hbm, amh_hbm, b2a_hbm, b2revb_hbm, out_hbm,
              ia_v, ir_v, arow_v, rrow_v, inp_v, out_v, sem_a, sem_r):
    wid = _wid()

    def chunk(c, carry):
        base = wid * BONDS_PER_W + c * MSG_CHUNK
        pltpu.sync_copy(b2a_hbm.at[pl.ds(base, MSG_CHUNK)], ia_v)
        pltpu.sync_copy(b2revb_hbm.at[pl.ds(base, MSG_CHUNK)], ir_v)
        cp_a = pltpu.async_copy(amh_hbm.at[ia_v], arow_v, sem_a)
        cp_r = pltpu.async_copy(h_hbm.at[ir_v], rrow_v, sem_r)
        pltpu.sync_copy(inp_hbm.at[pl.ds(base, MSG_CHUNK)], inp_v)
        cp_a.wait()
        cp_r.wait()

        def row(i, cc):
            for g in range(8):
                sl = pl.ds(g * 16, 16)
                v = inp_v[i, sl] + arow_v[i, sl] - rrow_v[i, sl]
                out_v[i, sl] = jnp.maximum(v, 0.0)
            return cc

        lax.fori_loop(0, MSG_CHUNK, row, 0)
        pltpu.sync_copy(out_v, out_hbm.at[pl.ds(base, MSG_CHUNK)])
        return carry

    lax.fori_loop(0, MSG_CHUNKS, chunk, 0)


IN_BLK = 1024


def _in_body(fb_ref, wi_ref, wh_ref, inp_ref, h_ref):
    inp = jnp.dot(fb_ref[...], wi_ref[...], preferred_element_type=jnp.float32)
    inp_ref[...] = inp
    h_ref[...] = jnp.dot(jnp.maximum(inp, 0.0), wh_ref[...],
                         preferred_element_type=jnp.float32)


def _input_stage(f_bonds, W_i, W_h):
    nblk = N_BONDS // IN_BLK
    return pl.pallas_call(
        _in_body,
        grid=(nblk,),
        in_specs=[
            pl.BlockSpec((IN_BLK, f_bonds.shape[1]), lambda i: (i, 0)),
            pl.BlockSpec((f_bonds.shape[1], HIDDEN), lambda i: (0, 0)),
            pl.BlockSpec((HIDDEN, HIDDEN), lambda i: (0, 0)),
        ],
        out_specs=[
            pl.BlockSpec((IN_BLK, HIDDEN), lambda i: (i, 0)),
            pl.BlockSpec((IN_BLK, HIDDEN), lambda i: (i, 0)),
        ],
        out_shape=[
            jax.ShapeDtypeStruct((N_BONDS, HIDDEN), jnp.float32),
            jax.ShapeDtypeStruct((N_BONDS, HIDDEN), jnp.float32),
        ],
    )(f_bonds, W_i, W_h)


def _mm_body(m_ref, wh_ref, h_ref):
    h_ref[...] = jnp.dot(m_ref[...], wh_ref[...],
                         preferred_element_type=jnp.float32)


def _h_stage(m, W_h):
    nblk = N_BONDS // IN_BLK
    return pl.pallas_call(
        _mm_body,
        grid=(nblk,),
        in_specs=[
            pl.BlockSpec((IN_BLK, HIDDEN), lambda i: (i, 0)),
            pl.BlockSpec((HIDDEN, HIDDEN), lambda i: (0, 0)),
        ],
        out_specs=pl.BlockSpec((IN_BLK, HIDDEN), lambda i: (i, 0)),
        out_shape=jax.ShapeDtypeStruct((N_BONDS, HIDDEN), jnp.float32),
    )(m, W_h)


OUT_BLK = 1000


def _out_body(fa_ref, am_ref, woa_ref, wom_ref, bo_ref, out_ref):
    acc = jnp.dot(fa_ref[...], woa_ref[...], preferred_element_type=jnp.float32)
    acc += jnp.dot(am_ref[...], wom_ref[...], preferred_element_type=jnp.float32)
    out_ref[...] = jnp.maximum(acc + bo_ref[...], 0.0)


def _out_stage(f_atoms, am, W_oa, W_om, b_o):
    nblk = N_ATOMS // OUT_BLK
    afd = f_atoms.shape[1]
    return pl.pallas_call(
        _out_body,
        grid=(nblk,),
        in_specs=[
            pl.BlockSpec((OUT_BLK, afd), lambda i: (i, 0)),
            pl.BlockSpec((OUT_BLK, HIDDEN), lambda i: (i, 0)),
            pl.BlockSpec((afd, HIDDEN), lambda i: (0, 0)),
            pl.BlockSpec((HIDDEN, HIDDEN), lambda i: (0, 0)),
            pl.BlockSpec((1, HIDDEN), lambda i: (0, 0)),
        ],
        out_specs=pl.BlockSpec((OUT_BLK, HIDDEN), lambda i: (i, 0)),
        out_shape=jax.ShapeDtypeStruct((N_ATOMS, HIDDEN), jnp.float32),
    )(f_atoms, am, W_oa, W_om, b_o)


def kernel(f_atoms, f_bonds, a2b, b2a, b2revb, W_i, W_h, W_o, b_o):
    a2b_flat = jnp.pad(
        a2b.astype(jnp.int32), ((0, A_PAD - N_ATOMS), (0, 0))
    ).reshape(-1)
    b2a = b2a.astype(jnp.int32)
    b2revb = b2revb.astype(jnp.int32)

    inp, h = _input_stage(f_bonds, W_i, W_h)
    for d in range(DEPTH - 1):
        amh = _seg_sum(h, a2b_flat)
        m = _msg_step(inp, h, amh, b2a, b2revb)
        if d < DEPTH - 2:
            h = _h_stage(m, W_h)
    am = _seg_sum(m, a2b_flat)[:N_ATOMS]
    afd = f_atoms.shape[1]
    return _out_stage(f_atoms, am, W_o[:afd], W_o[afd:], b_o.reshape(1, HIDDEN))


# speedup vs baseline: 1.3254x; 1.3254x over previous
import functools

import jax
import jax.numpy as jnp
from jax import lax
from jax.experimental import pallas as pl
from jax.experimental.pallas import tpu as pltpu
from jax.experimental.pallas import tpu_sc as plsc

N_ATOMS = 10000
N_BONDS = 320000
MAX_NB = 32
HIDDEN = 128
DEPTH = 3

NC = 2
NS = 16
NW = NC * NS

ATOMS_PER_W = 320
A_PAD = NW * ATOMS_PER_W
SEG_ATOMS_PER_CHUNK = 4
SEG_CHUNKS = ATOMS_PER_W // SEG_ATOMS_PER_CHUNK

BONDS_PER_W = N_BONDS // NW
MSG_CHUNK = 80
MSG_CHUNKS = BONDS_PER_W // MSG_CHUNK

_mesh = plsc.VectorSubcoreMesh(core_axis_name="c", subcore_axis_name="s")


def _wid():
    return lax.axis_index("s") * NC + lax.axis_index("c")


def _pipeline(n_chunks, issue, wait, compute, store_issue, store_wait):
    def body(c, slot, prefetch, wait_st):
        if prefetch:
            issue(c + 1, 1 - slot)
        wait(slot)
        if wait_st:
            store_wait(slot)
        compute(c, slot)
        store_issue(c, slot)

    issue(0, 0)
    body(0, 0, True, False)
    body(1, 1, True, False)
    pairs = (n_chunks - 3) // 2

    def pair(t, carry):
        c = 2 + 2 * t
        body(c, 0, True, True)
        body(c + 1, 1, True, True)
        return carry

    lax.fori_loop(0, pairs, pair, 0)
    for c in range(2 + 2 * pairs, n_chunks):
        body(c, c % 2, c < n_chunks - 1, True)
    store_wait(0)
    store_wait(1)


@functools.partial(
    pl.kernel,
    out_type=jax.ShapeDtypeStruct((A_PAD, HIDDEN), jnp.float32),
    mesh=_mesh,
    scratch_types=[
        pltpu.VMEM((ATOMS_PER_W * MAX_NB,), jnp.int32),
        pltpu.VMEM((2, 128, HIDDEN), jnp.float32),
        pltpu.VMEM((2, SEG_ATOMS_PER_CHUNK, HIDDEN), jnp.float32),
        pltpu.SemaphoreType.DMA,
        pltpu.SemaphoreType.DMA,
        pltpu.SemaphoreType.DMA,
        pltpu.SemaphoreType.DMA,
    ],
)
def _seg_sum(h_hbm, idx_hbm, out_hbm, idx_v, rows_v, acc_v,
             sem_g0, sem_g1, sem_s0, sem_s1):
    wid = _wid()
    sem_g = (sem_g0, sem_g1)
    sem_s = (sem_s0, sem_s1)
    pltpu.sync_copy(
        idx_hbm.at[pl.ds(wid * (ATOMS_PER_W * MAX_NB), ATOMS_PER_W * MAX_NB)],
        idx_v)

    def issue(c, slot):
        pltpu.async_copy(h_hbm.at[idx_v.at[pl.ds(c * 128, 128)]],
                         rows_v.at[slot], sem_g[slot])

    def wait(slot):
        pltpu.make_async_copy(h_hbm.at[pl.ds(0, 128)], rows_v.at[slot],
                              sem_g[slot]).wait()

    def compute(c, slot):
        for a in range(SEG_ATOMS_PER_CHUNK):
            def red(j, accs):
                return tuple(
                    accs[g] + rows_v[slot, a * MAX_NB + j, pl.ds(g * 16, 16)]
                    for g in range(8)
                )
            accs = tuple(jnp.zeros((16,), jnp.float32) for _ in range(8))
            accs = lax.fori_loop(0, MAX_NB, red, accs)
            for g in range(8):
                acc_v[slot, a, pl.ds(g * 16, 16)] = accs[g]

    def store_issue(c, slot):
        abase = wid * ATOMS_PER_W + c * SEG_ATOMS_PER_CHUNK
        pltpu.async_copy(acc_v.at[slot],
                         out_hbm.at[pl.ds(abase, SEG_ATOMS_PER_CHUNK)],
                         sem_s[slot])

    def store_wait(slot):
        pltpu.make_async_copy(acc_v.at[slot],
                              out_hbm.at[pl.ds(0, SEG_ATOMS_PER_CHUNK)],
                              sem_s[slot]).wait()

    _pipeline(SEG_CHUNKS, issue, wait, compute, store_issue, store_wait)


@functools.partial(
    pl.kernel,
    out_type=jax.ShapeDtypeStruct((N_BONDS, HIDDEN), jnp.float32),
    mesh=_mesh,
    scratch_types=[
        pltpu.VMEM((BONDS_PER_W,), jnp.int32),
        pltpu.VMEM((BONDS_PER_W,), jnp.int32),
        pltpu.VMEM((2, MSG_CHUNK, HIDDEN), jnp.float32),
        pltpu.VMEM((2, MSG_CHUNK, HIDDEN), jnp.float32),
        pltpu.VMEM((2, MSG_CHUNK, HIDDEN), jnp.float32),
        pltpu.VMEM((2, MSG_CHUNK, HIDDEN), jnp.float32),
        pltpu.SemaphoreType.DMA,
        pltpu.SemaphoreType.DMA,
        pltpu.SemaphoreType.DMA,
        pltpu.SemaphoreType.DMA,
    ],
)
def _msg_step(inp_hbm, h_hbm, amh_hbm, b2a_hbm, b2revb_hbm, out_hbm,
              ia_v, ir_v, arow_v, rrow_v, inp_v, out_v,
              sem_g0, sem_g1, sem_s0, sem_s1):
    wid = _wid()
    sem_g = (sem_g0, sem_g1)
    sem_s = (sem_s0, sem_s1)
    pltpu.sync_copy(b2a_hbm.at[pl.ds(wid * BONDS_PER_W, BONDS_PER_W)], ia_v)
    pltpu.sync_copy(b2revb_hbm.at[pl.ds(wid * BONDS_PER_W, BONDS_PER_W)], ir_v)

    def issue(c, slot):
        base = wid * BONDS_PER_W + c * MSG_CHUNK
        pltpu.async_copy(amh_hbm.at[ia_v.at[pl.ds(c * MSG_CHUNK, MSG_CHUNK)]],
                         arow_v.at[slot], sem_g[slot])
        pltpu.async_copy(h_hbm.at[ir_v.at[pl.ds(c * MSG_CHUNK, MSG_CHUNK)]],
                         rrow_v.at[slot], sem_g[slot])
        pltpu.async_copy(inp_hbm.at[pl.ds(base, MSG_CHUNK)],
                         inp_v.at[slot], sem_g[slot])

    def wait(slot):
        pltpu.make_async_copy(amh_hbm.at[pl.ds(0, MSG_CHUNK)],
                              arow_v.at[slot], sem_g[slot]).wait()
        pltpu.make_async_copy(h_hbm.at[pl.ds(0, MSG_CHUNK)],
                              rrow_v.at[slot], sem_g[slot]).wait()
        pltpu.make_async_copy(inp_hbm.at[pl.ds(0, MSG_CHUNK)],
                              inp_v.at[slot], sem_g[slot]).wait()

    def compute(c, slot):
        def row(i, cc):
            for g in range(8):
                sl = pl.ds(g * 16, 16)
                v = inp_v[slot, i, sl] + arow_v[slot, i, sl] - rrow_v[slot, i, sl]
                out_v[slot, i, sl] = jnp.maximum(v, 0.0)
            return cc

        lax.fori_loop(0, MSG_CHUNK, row, 0)

    def store_issue(c, slot):
        base = wid * BONDS_PER_W + c * MSG_CHUNK
        pltpu.async_copy(out_v.at[slot], out_hbm.at[pl.ds(base, MSG_CHUNK)],
                         sem_s[slot])

    def store_wait(slot):
        pltpu.make_async_copy(out_v.at[slot], out_hbm.at[pl.ds(0, MSG_CHUNK)],
                              sem_s[slot]).wait()

    _pipeline(MSG_CHUNKS, issue, wait, compute, store_issue, store_wait)


IN_BLK = 1024


def _in_body(fb_ref, wi_ref, wh_ref, inp_ref, h_ref):
    inp = jnp.dot(fb_ref[...], wi_ref[...], preferred_element_type=jnp.float32)
    inp_ref[...] = inp
    h_ref[...] = jnp.dot(jnp.maximum(inp, 0.0), wh_ref[...],
                         preferred_element_type=jnp.float32)


def _input_stage(f_bonds, W_i, W_h):
    nblk = N_BONDS // IN_BLK
    return pl.pallas_call(
        _in_body,
        grid=(nblk,),
        in_specs=[
            pl.BlockSpec((IN_BLK, f_bonds.shape[1]), lambda i: (i, 0)),
            pl.BlockSpec((f_bonds.shape[1], HIDDEN), lambda i: (0, 0)),
            pl.BlockSpec((HIDDEN, HIDDEN), lambda i: (0, 0)),
        ],
        out_specs=[
            pl.BlockSpec((IN_BLK, HIDDEN), lambda i: (i, 0)),
            pl.BlockSpec((IN_BLK, HIDDEN), lambda i: (i, 0)),
        ],
        out_shape=[
            jax.ShapeDtypeStruct((N_BONDS, HIDDEN), jnp.float32),
            jax.ShapeDtypeStruct((N_BONDS, HIDDEN), jnp.float32),
        ],
    )(f_bonds, W_i, W_h)


def _mm_body(m_ref, wh_ref, h_ref):
    h_ref[...] = jnp.dot(m_ref[...], wh_ref[...],
                         preferred_element_type=jnp.float32)


def _h_stage(m, W_h):
    nblk = N_BONDS // IN_BLK
    return pl.pallas_call(
        _mm_body,
        grid=(nblk,),
        in_specs=[
            pl.BlockSpec((IN_BLK, HIDDEN), lambda i: (i, 0)),
            pl.BlockSpec((HIDDEN, HIDDEN), lambda i: (0, 0)),
        ],
        out_specs=pl.BlockSpec((IN_BLK, HIDDEN), lambda i: (i, 0)),
        out_shape=jax.ShapeDtypeStruct((N_BONDS, HIDDEN), jnp.float32),
    )(m, W_h)


OUT_BLK = 1000


def _out_body(fa_ref, am_ref, woa_ref, wom_ref, bo_ref, out_ref):
    acc = jnp.dot(fa_ref[...], woa_ref[...], preferred_element_type=jnp.float32)
    acc += jnp.dot(am_ref[...], wom_ref[...], preferred_element_type=jnp.float32)
    out_ref[...] = jnp.maximum(acc + bo_ref[...], 0.0)


def _out_stage(f_atoms, am, W_oa, W_om, b_o):
    nblk = N_ATOMS // OUT_BLK
    afd = f_atoms.shape[1]
    return pl.pallas_call(
        _out_body,
        grid=(nblk,),
        in_specs=[
            pl.BlockSpec((OUT_BLK, afd), lambda i: (i, 0)),
            pl.BlockSpec((OUT_BLK, HIDDEN), lambda i: (i, 0)),
            pl.BlockSpec((afd, HIDDEN), lambda i: (0, 0)),
            pl.BlockSpec((HIDDEN, HIDDEN), lambda i: (0, 0)),
            pl.BlockSpec((1, HIDDEN), lambda i: (0, 0)),
        ],
        out_specs=pl.BlockSpec((OUT_BLK, HIDDEN), lambda i: (i, 0)),
        out_shape=jax.ShapeDtypeStruct((N_ATOMS, HIDDEN), jnp.float32),
    )(f_atoms, am, W_oa, W_om, b_o)


def kernel(f_atoms, f_bonds, a2b, b2a, b2revb, W_i, W_h, W_o, b_o):
    a2b_flat = jnp.pad(
        a2b.astype(jnp.int32), ((0, A_PAD - N_ATOMS), (0, 0))
    ).reshape(-1)
    b2a = b2a.astype(jnp.int32)
    b2revb = b2revb.astype(jnp.int32)

    inp, h = _input_stage(f_bonds, W_i, W_h)
    for d in range(DEPTH - 1):
        amh = _seg_sum(h, a2b_flat)
        m = _msg_step(inp, h, amh, b2a, b2revb)
        if d < DEPTH - 2:
            h = _h_stage(m, W_h)
    am = _seg_sum(m, a2b_flat)[:N_ATOMS]
    afd = f_atoms.shape[1]
    return _out_stage(f_atoms, am, W_o[:afd], W_o[afd:], b_o.reshape(1, HIDDEN))


# speedup vs baseline: 2.3290x; 1.7571x over previous
import functools

import jax
import jax.numpy as jnp
from jax import lax
from jax.experimental import pallas as pl
from jax.experimental.pallas import tpu as pltpu
from jax.experimental.pallas import tpu_sc as plsc

N_ATOMS = 10000
N_BONDS = 320000
MAX_NB = 32
HIDDEN = 128
DEPTH = 3

NC = 2
NS = 16
NW = NC * NS

ATOMS_PER_W = 320
A_PAD = NW * ATOMS_PER_W
SEG_ATOMS_PER_CHUNK = 8
SEG_CHUNKS = ATOMS_PER_W // SEG_ATOMS_PER_CHUNK
SEG_NBUF = 3

BONDS_PER_W = N_BONDS // NW
MSG_CHUNK = 80
MSG_CHUNKS = BONDS_PER_W // MSG_CHUNK

_mesh = plsc.VectorSubcoreMesh(core_axis_name="c", subcore_axis_name="s")


def _wid():
    return lax.axis_index("s") * NC + lax.axis_index("c")


def _pipeline(n_chunks, nbuf, issue, wait, compute, store_issue, store_wait):
    def body(c, slot, prefetch, wait_st):
        if prefetch:
            issue(c + nbuf - 1, (slot + nbuf - 1) % nbuf)
        wait(slot)
        if wait_st:
            store_wait(slot)
        compute(c, slot)
        store_issue(c, slot)

    for c in range(nbuf - 1):
        issue(c, c % nbuf)
    tail = nbuf - 1
    tail += (n_chunks - nbuf - tail) % nbuf
    for c in range(min(nbuf, n_chunks - tail)):
        body(c, c % nbuf, True, False)
    groups = (n_chunks - tail - nbuf) // nbuf

    def group(t, carry):
        c0 = nbuf + nbuf * t
        for k in range(nbuf):
            body(c0 + k, k % nbuf, True, True)
        return carry

    lax.fori_loop(0, groups, group, 0)
    for c in range(n_chunks - tail, n_chunks):
        body(c, c % nbuf, c + nbuf - 1 < n_chunks, True)
    for s in range(min(nbuf, n_chunks)):
        store_wait(s)


@functools.partial(
    pl.kernel,
    out_type=jax.ShapeDtypeStruct((A_PAD, HIDDEN), jnp.float32),
    mesh=_mesh,
    scratch_types=[
        pltpu.VMEM((ATOMS_PER_W * MAX_NB,), jnp.int32),
        pltpu.VMEM((SEG_NBUF, SEG_ATOMS_PER_CHUNK * MAX_NB, HIDDEN), jnp.float32),
        pltpu.VMEM((SEG_NBUF, SEG_ATOMS_PER_CHUNK, HIDDEN), jnp.float32),
        pltpu.SemaphoreType.DMA,
        pltpu.SemaphoreType.DMA,
        pltpu.SemaphoreType.DMA,
        pltpu.SemaphoreType.DMA,
        pltpu.SemaphoreType.DMA,
        pltpu.SemaphoreType.DMA,
    ],
)
def _seg_sum(h_hbm, idx_hbm, out_hbm, idx_v, rows_v, acc_v,
             sem_g0, sem_g1, sem_g2, sem_s0, sem_s1, sem_s2):
    wid = _wid()
    sem_g = (sem_g0, sem_g1, sem_g2)
    sem_s = (sem_s0, sem_s1, sem_s2)
    pltpu.sync_copy(
        idx_hbm.at[pl.ds(wid * (ATOMS_PER_W * MAX_NB), ATOMS_PER_W * MAX_NB)],
        idx_v)
    half = SEG_ATOMS_PER_CHUNK * MAX_NB // 2

    def issue(c, slot):
        ib = c * SEG_ATOMS_PER_CHUNK * MAX_NB
        pltpu.async_copy(h_hbm.at[idx_v.at[pl.ds(ib, half)]],
                         rows_v.at[slot].at[pl.ds(0, half)], sem_g[slot])
        pltpu.async_copy(h_hbm.at[idx_v.at[pl.ds(ib + half, half)]],
                         rows_v.at[slot].at[pl.ds(half, half)], sem_g[slot])

    def wait(slot):
        pltpu.make_async_copy(h_hbm.at[pl.ds(0, half)],
                              rows_v.at[slot].at[pl.ds(0, half)],
                              sem_g[slot]).wait()
        pltpu.make_async_copy(h_hbm.at[pl.ds(0, half)],
                              rows_v.at[slot].at[pl.ds(half, half)],
                              sem_g[slot]).wait()

    def compute(c, slot):
        for a in range(SEG_ATOMS_PER_CHUNK):
            def red(j, accs):
                return tuple(
                    accs[g] + rows_v[slot, a * MAX_NB + j, pl.ds(g * 16, 16)]
                    for g in range(8)
                )
            accs = tuple(jnp.zeros((16,), jnp.float32) for _ in range(8))
            accs = lax.fori_loop(0, MAX_NB, red, accs)
            for g in range(8):
                acc_v[slot, a, pl.ds(g * 16, 16)] = accs[g]

    def store_issue(c, slot):
        abase = wid * ATOMS_PER_W + c * SEG_ATOMS_PER_CHUNK
        pltpu.async_copy(acc_v.at[slot],
                         out_hbm.at[pl.ds(abase, SEG_ATOMS_PER_CHUNK)],
                         sem_s[slot])

    def store_wait(slot):
        pltpu.make_async_copy(acc_v.at[slot],
                              out_hbm.at[pl.ds(0, SEG_ATOMS_PER_CHUNK)],
                              sem_s[slot]).wait()

    _pipeline(SEG_CHUNKS, SEG_NBUF, issue, wait, compute, store_issue,
              store_wait)


@functools.partial(
    pl.kernel,
    out_type=jax.ShapeDtypeStruct((N_BONDS, HIDDEN), jnp.float32),
    mesh=_mesh,
    scratch_types=[
        pltpu.VMEM((BONDS_PER_W,), jnp.int32),
        pltpu.VMEM((BONDS_PER_W,), jnp.int32),
        pltpu.VMEM((2, MSG_CHUNK, HIDDEN), jnp.float32),
        pltpu.VMEM((2, MSG_CHUNK, HIDDEN), jnp.float32),
        pltpu.VMEM((2, MSG_CHUNK, HIDDEN), jnp.float32),
        pltpu.VMEM((2, MSG_CHUNK, HIDDEN), jnp.float32),
        pltpu.SemaphoreType.DMA,
        pltpu.SemaphoreType.DMA,
        pltpu.SemaphoreType.DMA,
        pltpu.SemaphoreType.DMA,
    ],
)
def _msg_step(inp_hbm, h_hbm, amh_hbm, b2a_hbm, b2revb_hbm, out_hbm,
              ia_v, ir_v, arow_v, rrow_v, inp_v, out_v,
              sem_g0, sem_g1, sem_s0, sem_s1):
    wid = _wid()
    sem_g = (sem_g0, sem_g1)
    sem_s = (sem_s0, sem_s1)
    pltpu.sync_copy(b2a_hbm.at[pl.ds(wid * BONDS_PER_W, BONDS_PER_W)], ia_v)
    pltpu.sync_copy(b2revb_hbm.at[pl.ds(wid * BONDS_PER_W, BONDS_PER_W)], ir_v)

    def issue(c, slot):
        base = wid * BONDS_PER_W + c * MSG_CHUNK
        pltpu.async_copy(amh_hbm.at[ia_v.at[pl.ds(c * MSG_CHUNK, MSG_CHUNK)]],
                         arow_v.at[slot], sem_g[slot])
        pltpu.async_copy(h_hbm.at[ir_v.at[pl.ds(c * MSG_CHUNK, MSG_CHUNK)]],
                         rrow_v.at[slot], sem_g[slot])
        pltpu.async_copy(inp_hbm.at[pl.ds(base, MSG_CHUNK)],
                         inp_v.at[slot], sem_g[slot])

    def wait(slot):
        pltpu.make_async_copy(amh_hbm.at[pl.ds(0, MSG_CHUNK)],
                              arow_v.at[slot], sem_g[slot]).wait()
        pltpu.make_async_copy(h_hbm.at[pl.ds(0, MSG_CHUNK)],
                              rrow_v.at[slot], sem_g[slot]).wait()
        pltpu.make_async_copy(inp_hbm.at[pl.ds(0, MSG_CHUNK)],
                              inp_v.at[slot], sem_g[slot]).wait()

    def compute(c, slot):
        def row(i, cc):
            for g in range(8):
                sl = pl.ds(g * 16, 16)
                v = inp_v[slot, i, sl] + arow_v[slot, i, sl] - rrow_v[slot, i, sl]
                out_v[slot, i, sl] = jnp.maximum(v, 0.0)
            return cc

        lax.fori_loop(0, MSG_CHUNK, row, 0)

    def store_issue(c, slot):
        base = wid * BONDS_PER_W + c * MSG_CHUNK
        pltpu.async_copy(out_v.at[slot], out_hbm.at[pl.ds(base, MSG_CHUNK)],
                         sem_s[slot])

    def store_wait(slot):
        pltpu.make_async_copy(out_v.at[slot], out_hbm.at[pl.ds(0, MSG_CHUNK)],
                              sem_s[slot]).wait()

    _pipeline(MSG_CHUNKS, 2, issue, wait, compute, store_issue, store_wait)


IN_BLK = 1024


def _in_body(fb_ref, wi_ref, wh_ref, inp_ref, h_ref):
    inp = jnp.dot(fb_ref[...], wi_ref[...], preferred_element_type=jnp.float32)
    inp_ref[...] = inp
    h_ref[...] = jnp.dot(jnp.maximum(inp, 0.0), wh_ref[...],
                         preferred_element_type=jnp.float32)


def _input_stage(f_bonds, W_i, W_h):
    nblk = N_BONDS // IN_BLK
    return pl.pallas_call(
        _in_body,
        grid=(nblk,),
        in_specs=[
            pl.BlockSpec((IN_BLK, f_bonds.shape[1]), lambda i: (i, 0)),
            pl.BlockSpec((f_bonds.shape[1], HIDDEN), lambda i: (0, 0)),
            pl.BlockSpec((HIDDEN, HIDDEN), lambda i: (0, 0)),
        ],
        out_specs=[
            pl.BlockSpec((IN_BLK, HIDDEN), lambda i: (i, 0)),
            pl.BlockSpec((IN_BLK, HIDDEN), lambda i: (i, 0)),
        ],
        out_shape=[
            jax.ShapeDtypeStruct((N_BONDS, HIDDEN), jnp.float32),
            jax.ShapeDtypeStruct((N_BONDS, HIDDEN), jnp.float32),
        ],
    )(f_bonds, W_i, W_h)


def _mm_body(m_ref, wh_ref, h_ref):
    h_ref[...] = jnp.dot(m_ref[...], wh_ref[...],
                         preferred_element_type=jnp.float32)


def _h_stage(m, W_h):
    nblk = N_BONDS // IN_BLK
    return pl.pallas_call(
        _mm_body,
        grid=(nblk,),
        in_specs=[
            pl.BlockSpec((IN_BLK, HIDDEN), lambda i: (i, 0)),
            pl.BlockSpec((HIDDEN, HIDDEN), lambda i: (0, 0)),
        ],
        out_specs=pl.BlockSpec((IN_BLK, HIDDEN), lambda i: (i, 0)),
        out_shape=jax.ShapeDtypeStruct((N_BONDS, HIDDEN), jnp.float32),
    )(m, W_h)


OUT_BLK = 1000


def _out_body(fa_ref, am_ref, woa_ref, wom_ref, bo_ref, out_ref):
    acc = jnp.dot(fa_ref[...], woa_ref[...], preferred_element_type=jnp.float32)
    acc += jnp.dot(am_ref[...], wom_ref[...], preferred_element_type=jnp.float32)
    out_ref[...] = jnp.maximum(acc + bo_ref[...], 0.0)


def _out_stage(f_atoms, am, W_oa, W_om, b_o):
    nblk = N_ATOMS // OUT_BLK
    afd = f_atoms.shape[1]
    return pl.pallas_call(
        _out_body,
        grid=(nblk,),
        in_specs=[
            pl.BlockSpec((OUT_BLK, afd), lambda i: (i, 0)),
            pl.BlockSpec((OUT_BLK, HIDDEN), lambda i: (i, 0)),
            pl.BlockSpec((afd, HIDDEN), lambda i: (0, 0)),
            pl.BlockSpec((HIDDEN, HIDDEN), lambda i: (0, 0)),
            pl.BlockSpec((1, HIDDEN), lambda i: (0, 0)),
        ],
        out_specs=pl.BlockSpec((OUT_BLK, HIDDEN), lambda i: (i, 0)),
        out_shape=jax.ShapeDtypeStruct((N_ATOMS, HIDDEN), jnp.float32),
    )(f_atoms, am, W_oa, W_om, b_o)


def kernel(f_atoms, f_bonds, a2b, b2a, b2revb, W_i, W_h, W_o, b_o):
    pad_idx = (jnp.arange((A_PAD - N_ATOMS) * MAX_NB, dtype=jnp.int32)
               % N_BONDS).reshape(A_PAD - N_ATOMS, MAX_NB)
    a2b_flat = jnp.concatenate(
        [a2b.astype(jnp.int32), pad_idx], axis=0).reshape(-1)
    b2a = b2a.astype(jnp.int32)
    b2revb = b2revb.astype(jnp.int32)

    inp, h = _input_stage(f_bonds, W_i, W_h)
    for d in range(DEPTH - 1):
        amh = _seg_sum(h, a2b_flat)
        m = _msg_step(inp, h, amh, b2a, b2revb)
        if d < DEPTH - 2:
            h = _h_stage(m, W_h)
    am = _seg_sum(m, a2b_flat)[:N_ATOMS]
    afd = f_atoms.shape[1]
    return _out_stage(f_atoms, am, W_o[:afd], W_o[afd:], b_o.reshape(1, HIDDEN))


# speedup vs baseline: 3.3210x; 1.4259x over previous
import functools

import jax
import jax.numpy as jnp
from jax import lax
from jax.experimental import pallas as pl
from jax.experimental.pallas import tpu as pltpu
from jax.experimental.pallas import tpu_sc as plsc

N_ATOMS = 10000
N_BONDS = 320000
MAX_NB = 32
HIDDEN = 128
DEPTH = 3

NC = 2
NS = 16
NW = NC * NS

ATOMS_PER_W = 320
A_PAD = NW * ATOMS_PER_W
SEG_ATOMS_PER_CHUNK = 8
SEG_CHUNKS = ATOMS_PER_W // SEG_ATOMS_PER_CHUNK
SEG_NBUF = 3

BONDS_PER_W = N_BONDS // NW
MSG_CHUNK = 80
MSG_CHUNKS = BONDS_PER_W // MSG_CHUNK

_mesh = plsc.VectorSubcoreMesh(core_axis_name="c", subcore_axis_name="s")


def _wid():
    return lax.axis_index("s") * NC + lax.axis_index("c")


def _pipeline(n_chunks, nbuf, issue, wait, compute, store_issue, store_wait):
    def body(c, slot, prefetch, wait_st):
        if prefetch:
            issue(c + nbuf - 1, (slot + nbuf - 1) % nbuf)
        wait(slot)
        if wait_st:
            store_wait(slot)
        compute(c, slot)
        store_issue(c, slot)

    for c in range(nbuf - 1):
        issue(c, c % nbuf)
    tail = nbuf - 1
    tail += (n_chunks - nbuf - tail) % nbuf
    for c in range(min(nbuf, n_chunks - tail)):
        body(c, c % nbuf, True, False)
    groups = (n_chunks - tail - nbuf) // nbuf

    def group(t, carry):
        c0 = nbuf + nbuf * t
        for k in range(nbuf):
            body(c0 + k, k % nbuf, True, True)
        return carry

    lax.fori_loop(0, groups, group, 0)
    for c in range(n_chunks - tail, n_chunks):
        body(c, c % nbuf, c + nbuf - 1 < n_chunks, True)
    for s in range(min(nbuf, n_chunks)):
        store_wait(s)


@functools.partial(
    pl.kernel,
    out_type=jax.ShapeDtypeStruct((A_PAD, HIDDEN), jnp.float32),
    mesh=_mesh,
    scratch_types=[
        pltpu.VMEM((ATOMS_PER_W * MAX_NB,), jnp.int32),
        pltpu.VMEM((SEG_NBUF, SEG_ATOMS_PER_CHUNK * MAX_NB, HIDDEN), jnp.float32),
        pltpu.VMEM((SEG_NBUF, SEG_ATOMS_PER_CHUNK, HIDDEN), jnp.float32),
        pltpu.SemaphoreType.DMA,
        pltpu.SemaphoreType.DMA,
        pltpu.SemaphoreType.DMA,
        pltpu.SemaphoreType.DMA,
        pltpu.SemaphoreType.DMA,
        pltpu.SemaphoreType.DMA,
    ],
)
def _seg_sum(h_hbm, idx_hbm, out_hbm, idx_v, rows_v, acc_v,
             sem_g0, sem_g1, sem_g2, sem_s0, sem_s1, sem_s2):
    wid = _wid()
    sem_g = (sem_g0, sem_g1, sem_g2)
    sem_s = (sem_s0, sem_s1, sem_s2)
    pltpu.sync_copy(
        idx_hbm.at[pl.ds(wid * (ATOMS_PER_W * MAX_NB), ATOMS_PER_W * MAX_NB)],
        idx_v)
    half = SEG_ATOMS_PER_CHUNK * MAX_NB // 2

    def issue(c, slot):
        ib = c * SEG_ATOMS_PER_CHUNK * MAX_NB
        pltpu.async_copy(h_hbm.at[idx_v.at[pl.ds(ib, half)]],
                         rows_v.at[slot].at[pl.ds(0, half)], sem_g[slot])
        pltpu.async_copy(h_hbm.at[idx_v.at[pl.ds(ib + half, half)]],
                         rows_v.at[slot].at[pl.ds(half, half)], sem_g[slot])

    def wait(slot):
        pltpu.make_async_copy(h_hbm.at[pl.ds(0, half)],
                              rows_v.at[slot].at[pl.ds(0, half)],
                              sem_g[slot]).wait()
        pltpu.make_async_copy(h_hbm.at[pl.ds(0, half)],
                              rows_v.at[slot].at[pl.ds(half, half)],
                              sem_g[slot]).wait()

    def compute(c, slot):
        for a in range(SEG_ATOMS_PER_CHUNK):
            def red(j, accs):
                return tuple(
                    accs[g] + rows_v[slot, a * MAX_NB + j, pl.ds(g * 16, 16)]
                    for g in range(8)
                )
            accs = tuple(jnp.zeros((16,), jnp.float32) for _ in range(8))
            accs = lax.fori_loop(0, MAX_NB, red, accs)
            for g in range(8):
                acc_v[slot, a, pl.ds(g * 16, 16)] = accs[g]

    def store_issue(c, slot):
        abase = wid * ATOMS_PER_W + c * SEG_ATOMS_PER_CHUNK
        pltpu.async_copy(acc_v.at[slot],
                         out_hbm.at[pl.ds(abase, SEG_ATOMS_PER_CHUNK)],
                         sem_s[slot])

    def store_wait(slot):
        pltpu.make_async_copy(acc_v.at[slot],
                              out_hbm.at[pl.ds(0, SEG_ATOMS_PER_CHUNK)],
                              sem_s[slot]).wait()

    _pipeline(SEG_CHUNKS, SEG_NBUF, issue, wait, compute, store_issue,
              store_wait)


@functools.partial(
    pl.kernel,
    out_type=jax.ShapeDtypeStruct((N_BONDS, HIDDEN), jnp.float32),
    mesh=_mesh,
    scratch_types=[
        pltpu.VMEM((BONDS_PER_W,), jnp.int32),
        pltpu.VMEM((BONDS_PER_W,), jnp.int32),
        pltpu.VMEM((2, MSG_CHUNK, HIDDEN), jnp.float32),
        pltpu.VMEM((2, MSG_CHUNK, HIDDEN), jnp.float32),
        pltpu.VMEM((2, MSG_CHUNK, HIDDEN), jnp.float32),
        pltpu.VMEM((2, MSG_CHUNK, HIDDEN), jnp.float32),
        pltpu.SemaphoreType.DMA,
        pltpu.SemaphoreType.DMA,
        pltpu.SemaphoreType.DMA,
        pltpu.SemaphoreType.DMA,
    ],
)
def _msg_step(inp_hbm, h_hbm, amh_hbm, b2a_hbm, b2revb_hbm, out_hbm,
              ia_v, ir_v, arow_v, rrow_v, inp_v, out_v,
              sem_g0, sem_g1, sem_s0, sem_s1):
    wid = _wid()
    sem_g = (sem_g0, sem_g1)
    sem_s = (sem_s0, sem_s1)
    pltpu.sync_copy(b2a_hbm.at[pl.ds(wid * BONDS_PER_W, BONDS_PER_W)], ia_v)
    pltpu.sync_copy(b2revb_hbm.at[pl.ds(wid * BONDS_PER_W, BONDS_PER_W)], ir_v)

    def issue(c, slot):
        base = wid * BONDS_PER_W + c * MSG_CHUNK
        pltpu.async_copy(amh_hbm.at[ia_v.at[pl.ds(c * MSG_CHUNK, MSG_CHUNK)]],
                         arow_v.at[slot], sem_g[slot])
        pltpu.async_copy(h_hbm.at[ir_v.at[pl.ds(c * MSG_CHUNK, MSG_CHUNK)]],
                         rrow_v.at[slot], sem_g[slot])
        pltpu.async_copy(inp_hbm.at[pl.ds(base, MSG_CHUNK)],
                         inp_v.at[slot], sem_g[slot])

    def wait(slot):
        pltpu.make_async_copy(amh_hbm.at[pl.ds(0, MSG_CHUNK)],
                              arow_v.at[slot], sem_g[slot]).wait()
        pltpu.make_async_copy(h_hbm.at[pl.ds(0, MSG_CHUNK)],
                              rrow_v.at[slot], sem_g[slot]).wait()
        pltpu.make_async_copy(inp_hbm.at[pl.ds(0, MSG_CHUNK)],
                              inp_v.at[slot], sem_g[slot]).wait()

    def compute(c, slot):
        def row(i, cc):
            for g in range(8):
                sl = pl.ds(g * 16, 16)
                v = inp_v[slot, i, sl] + arow_v[slot, i, sl] - rrow_v[slot, i, sl]
                out_v[slot, i, sl] = jnp.maximum(v, 0.0)
            return cc

        lax.fori_loop(0, MSG_CHUNK, row, 0)

    def store_issue(c, slot):
        base = wid * BONDS_PER_W + c * MSG_CHUNK
        pltpu.async_copy(out_v.at[slot], out_hbm.at[pl.ds(base, MSG_CHUNK)],
                         sem_s[slot])

    def store_wait(slot):
        pltpu.make_async_copy(out_v.at[slot], out_hbm.at[pl.ds(0, MSG_CHUNK)],
                              sem_s[slot]).wait()

    _pipeline(MSG_CHUNKS, 2, issue, wait, compute, store_issue, store_wait)


IN_BLK = 2560


def _in_body(fbt_ref, wi_ref, wh_ref, inp_ref, h_ref):
    inp = jax.lax.dot_general(
        fbt_ref[...], wi_ref[...], (((0,), (0,)), ((), ())),
        preferred_element_type=jnp.float32)
    inp_ref[...] = inp
    h_ref[...] = jnp.dot(jnp.maximum(inp, 0.0), wh_ref[...],
                         preferred_element_type=jnp.float32)


def _input_stage(f_bonds_t, W_i, W_h):
    nblk = N_BONDS // IN_BLK
    bfd = f_bonds_t.shape[0]
    return pl.pallas_call(
        _in_body,
        grid=(nblk,),
        in_specs=[
            pl.BlockSpec((bfd, IN_BLK), lambda i: (0, i)),
            pl.BlockSpec((bfd, HIDDEN), lambda i: (0, 0)),
            pl.BlockSpec((HIDDEN, HIDDEN), lambda i: (0, 0)),
        ],
        out_specs=[
            pl.BlockSpec((IN_BLK, HIDDEN), lambda i: (i, 0)),
            pl.BlockSpec((IN_BLK, HIDDEN), lambda i: (i, 0)),
        ],
        out_shape=[
            jax.ShapeDtypeStruct((N_BONDS, HIDDEN), jnp.float32),
            jax.ShapeDtypeStruct((N_BONDS, HIDDEN), jnp.float32),
        ],
    )(f_bonds_t, W_i, W_h)


def _mm_body(m_ref, wh_ref, h_ref):
    h_ref[...] = jnp.dot(m_ref[...], wh_ref[...],
                         preferred_element_type=jnp.float32)


def _h_stage(m, W_h):
    nblk = N_BONDS // IN_BLK
    return pl.pallas_call(
        _mm_body,
        grid=(nblk,),
        in_specs=[
            pl.BlockSpec((IN_BLK, HIDDEN), lambda i: (i, 0)),
            pl.BlockSpec((HIDDEN, HIDDEN), lambda i: (0, 0)),
        ],
        out_specs=pl.BlockSpec((IN_BLK, HIDDEN), lambda i: (i, 0)),
        out_shape=jax.ShapeDtypeStruct((N_BONDS, HIDDEN), jnp.float32),
    )(m, W_h)


OUT_BLK = 2000


def _out_body(fa_ref, am_ref, woa_ref, wom_ref, bo_ref, out_ref):
    acc = jnp.dot(fa_ref[...], woa_ref[...], preferred_element_type=jnp.float32)
    acc += jnp.dot(am_ref[...], wom_ref[...], preferred_element_type=jnp.float32)
    out_ref[...] = jnp.maximum(acc + bo_ref[...], 0.0)


def _out_stage(f_atoms, am, W_oa, W_om, b_o):
    nblk = N_ATOMS // OUT_BLK
    afd = f_atoms.shape[1]
    return pl.pallas_call(
        _out_body,
        grid=(nblk,),
        in_specs=[
            pl.BlockSpec((OUT_BLK, afd), lambda i: (i, 0)),
            pl.BlockSpec((OUT_BLK, HIDDEN), lambda i: (i, 0)),
            pl.BlockSpec((afd, HIDDEN), lambda i: (0, 0)),
            pl.BlockSpec((HIDDEN, HIDDEN), lambda i: (0, 0)),
            pl.BlockSpec((1, HIDDEN), lambda i: (0, 0)),
        ],
        out_specs=pl.BlockSpec((OUT_BLK, HIDDEN), lambda i: (i, 0)),
        out_shape=jax.ShapeDtypeStruct((N_ATOMS, HIDDEN), jnp.float32),
    )(f_atoms, am, W_oa, W_om, b_o)


def kernel(f_atoms, f_bonds, a2b, b2a, b2revb, W_i, W_h, W_o, b_o):
    pad_idx = (jnp.arange((A_PAD - N_ATOMS) * MAX_NB, dtype=jnp.int32)
               % N_BONDS).reshape(A_PAD - N_ATOMS, MAX_NB)
    a2b_flat = jnp.concatenate(
        [a2b.astype(jnp.int32), pad_idx], axis=0).reshape(-1)
    b2a = b2a.astype(jnp.int32)
    b2revb = b2revb.astype(jnp.int32)

    inp, h = _input_stage(f_bonds.T, W_i, W_h)
    for d in range(DEPTH - 1):
        amh = _seg_sum(h, a2b_flat)
        m = _msg_step(inp, h, amh, b2a, b2revb)
        if d < DEPTH - 2:
            h = _h_stage(m, W_h)
    am = _seg_sum(m, a2b_flat)[:N_ATOMS]
    afd = f_atoms.shape[1]
    return _out_stage(f_atoms, am, W_o[:afd], W_o[afd:],
                      b_o.reshape(1, HIDDEN))


# speedup vs baseline: 3.6398x; 1.0960x over previous
import functools

import jax
import jax.numpy as jnp
from jax import lax
from jax.experimental import pallas as pl
from jax.experimental.pallas import tpu as pltpu
from jax.experimental.pallas import tpu_sc as plsc

N_ATOMS = 10000
N_BONDS = 320000
MAX_NB = 32
HIDDEN = 128
DEPTH = 3

NC = 2
NS = 16
NW = NC * NS

ATOMS_PER_W = 320
A_PAD = NW * ATOMS_PER_W
SEG_ATOMS_PER_CHUNK = 8
SEG_CHUNKS = ATOMS_PER_W // SEG_ATOMS_PER_CHUNK
SEG_NBUF = 3

BONDS_PER_W = N_BONDS // NW
MSG_CHUNK = 80
MSG_CHUNKS = BONDS_PER_W // MSG_CHUNK

_mesh = plsc.VectorSubcoreMesh(core_axis_name="c", subcore_axis_name="s")


def _wid():
    return lax.axis_index("s") * NC + lax.axis_index("c")


def _pipeline(n_chunks, nbuf, issue, wait, compute, store_issue, store_wait):
    def body(c, slot, prefetch, wait_st):
        if prefetch:
            issue(c + nbuf - 1, (slot + nbuf - 1) % nbuf)
        wait(slot)
        if wait_st:
            store_wait(slot)
        compute(c, slot)
        store_issue(c, slot)

    for c in range(nbuf - 1):
        issue(c, c % nbuf)
    tail = nbuf - 1
    tail += (n_chunks - nbuf - tail) % nbuf
    for c in range(min(nbuf, n_chunks - tail)):
        body(c, c % nbuf, True, False)
    groups = (n_chunks - tail - nbuf) // nbuf

    def group(t, carry):
        c0 = nbuf + nbuf * t
        for k in range(nbuf):
            body(c0 + k, k % nbuf, True, True)
        return carry

    lax.fori_loop(0, groups, group, 0)
    for c in range(n_chunks - tail, n_chunks):
        body(c, c % nbuf, c + nbuf - 1 < n_chunks, True)
    for s in range(min(nbuf, n_chunks)):
        store_wait(s)


@functools.partial(
    pl.kernel,
    out_type=jax.ShapeDtypeStruct((A_PAD, HIDDEN), jnp.float32),
    mesh=_mesh,
    scratch_types=[
        pltpu.VMEM((ATOMS_PER_W * MAX_NB,), jnp.int32),
        pltpu.VMEM((SEG_NBUF, SEG_ATOMS_PER_CHUNK * MAX_NB, HIDDEN), jnp.float32),
        pltpu.VMEM((SEG_NBUF, SEG_ATOMS_PER_CHUNK, HIDDEN), jnp.float32),
        pltpu.SemaphoreType.DMA,
        pltpu.SemaphoreType.DMA,
        pltpu.SemaphoreType.DMA,
        pltpu.SemaphoreType.DMA,
        pltpu.SemaphoreType.DMA,
        pltpu.SemaphoreType.DMA,
    ],
)
def _seg_sum(h_hbm, idx_hbm, out_hbm, idx_v, rows_v, acc_v,
             sem_g0, sem_g1, sem_g2, sem_s0, sem_s1, sem_s2):
    wid = _wid()
    sem_g = (sem_g0, sem_g1, sem_g2)
    sem_s = (sem_s0, sem_s1, sem_s2)
    pltpu.sync_copy(
        idx_hbm.at[pl.ds(wid * (ATOMS_PER_W * MAX_NB), ATOMS_PER_W * MAX_NB)],
        idx_v)
    half = SEG_ATOMS_PER_CHUNK * MAX_NB // 2

    def issue(c, slot):
        ib = c * SEG_ATOMS_PER_CHUNK * MAX_NB
        pltpu.async_copy(h_hbm.at[idx_v.at[pl.ds(ib, half)]],
                         rows_v.at[slot].at[pl.ds(0, half)], sem_g[slot])
        pltpu.async_copy(h_hbm.at[idx_v.at[pl.ds(ib + half, half)]],
                         rows_v.at[slot].at[pl.ds(half, half)], sem_g[slot])

    def wait(slot):
        pltpu.make_async_copy(h_hbm.at[pl.ds(0, half)],
                              rows_v.at[slot].at[pl.ds(0, half)],
                              sem_g[slot]).wait()
        pltpu.make_async_copy(h_hbm.at[pl.ds(0, half)],
                              rows_v.at[slot].at[pl.ds(half, half)],
                              sem_g[slot]).wait()

    def compute(c, slot):
        for a in range(SEG_ATOMS_PER_CHUNK):
            def red(j, accs):
                return tuple(
                    accs[g] + rows_v[slot, a * MAX_NB + j, pl.ds(g * 16, 16)]
                    for g in range(8)
                )
            accs = tuple(jnp.zeros((16,), jnp.float32) for _ in range(8))
            accs = lax.fori_loop(0, MAX_NB, red, accs)
            for g in range(8):
                acc_v[slot, a, pl.ds(g * 16, 16)] = accs[g]

    def store_issue(c, slot):
        abase = wid * ATOMS_PER_W + c * SEG_ATOMS_PER_CHUNK
        pltpu.async_copy(acc_v.at[slot],
                         out_hbm.at[pl.ds(abase, SEG_ATOMS_PER_CHUNK)],
                         sem_s[slot])

    def store_wait(slot):
        pltpu.make_async_copy(acc_v.at[slot],
                              out_hbm.at[pl.ds(0, SEG_ATOMS_PER_CHUNK)],
                              sem_s[slot]).wait()

    _pipeline(SEG_CHUNKS, SEG_NBUF, issue, wait, compute, store_issue,
              store_wait)


@functools.partial(
    pl.kernel,
    out_type=jax.ShapeDtypeStruct((N_BONDS, HIDDEN), jnp.float32),
    mesh=_mesh,
    scratch_types=[
        pltpu.VMEM((BONDS_PER_W,), jnp.int32),
        pltpu.VMEM((BONDS_PER_W,), jnp.int32),
        pltpu.VMEM((2, MSG_CHUNK, HIDDEN), jnp.float32),
        pltpu.VMEM((2, MSG_CHUNK, HIDDEN), jnp.float32),
        pltpu.VMEM((2, MSG_CHUNK, HIDDEN), jnp.float32),
        pltpu.VMEM((2, MSG_CHUNK, HIDDEN), jnp.float32),
        pltpu.SemaphoreType.DMA,
        pltpu.SemaphoreType.DMA,
        pltpu.SemaphoreType.DMA,
        pltpu.SemaphoreType.DMA,
    ],
)
def _msg_step(inp_hbm, h_hbm, amh_hbm, b2a_hbm, b2revb_hbm, out_hbm,
              ia_v, ir_v, arow_v, rrow_v, inp_v, out_v,
              sem_g0, sem_g1, sem_s0, sem_s1):
    wid = _wid()
    sem_g = (sem_g0, sem_g1)
    sem_s = (sem_s0, sem_s1)
    pltpu.sync_copy(b2a_hbm.at[pl.ds(wid * BONDS_PER_W, BONDS_PER_W)], ia_v)
    pltpu.sync_copy(b2revb_hbm.at[pl.ds(wid * BONDS_PER_W, BONDS_PER_W)], ir_v)

    def issue(c, slot):
        base = wid * BONDS_PER_W + c * MSG_CHUNK
        pltpu.async_copy(amh_hbm.at[ia_v.at[pl.ds(c * MSG_CHUNK, MSG_CHUNK)]],
                         arow_v.at[slot], sem_g[slot])
        pltpu.async_copy(h_hbm.at[ir_v.at[pl.ds(c * MSG_CHUNK, MSG_CHUNK)]],
                         rrow_v.at[slot], sem_g[slot])
        pltpu.async_copy(inp_hbm.at[pl.ds(base, MSG_CHUNK)],
                         inp_v.at[slot], sem_g[slot])

    def wait(slot):
        pltpu.make_async_copy(amh_hbm.at[pl.ds(0, MSG_CHUNK)],
                              arow_v.at[slot], sem_g[slot]).wait()
        pltpu.make_async_copy(h_hbm.at[pl.ds(0, MSG_CHUNK)],
                              rrow_v.at[slot], sem_g[slot]).wait()
        pltpu.make_async_copy(inp_hbm.at[pl.ds(0, MSG_CHUNK)],
                              inp_v.at[slot], sem_g[slot]).wait()

    def compute(c, slot):
        def row(i, cc):
            for g in range(8):
                sl = pl.ds(g * 16, 16)
                v = inp_v[slot, i, sl] + arow_v[slot, i, sl] - rrow_v[slot, i, sl]
                out_v[slot, i, sl] = jnp.maximum(v, 0.0)
            return cc

        lax.fori_loop(0, MSG_CHUNK, row, 0)

    def store_issue(c, slot):
        base = wid * BONDS_PER_W + c * MSG_CHUNK
        pltpu.async_copy(out_v.at[slot], out_hbm.at[pl.ds(base, MSG_CHUNK)],
                         sem_s[slot])

    def store_wait(slot):
        pltpu.make_async_copy(out_v.at[slot], out_hbm.at[pl.ds(0, MSG_CHUNK)],
                              sem_s[slot]).wait()

    _pipeline(MSG_CHUNKS, 2, issue, wait, compute, store_issue, store_wait)


IN_BLK = 6400


def _in_body(fbt_ref, wi_ref, wh_ref, inp_ref, h_ref):
    inp = jax.lax.dot_general(
        fbt_ref[...], wi_ref[...], (((0,), (0,)), ((), ())),
        preferred_element_type=jnp.float32)
    inp_ref[...] = inp
    h_ref[...] = jnp.dot(jnp.maximum(inp, 0.0), wh_ref[...],
                         preferred_element_type=jnp.float32)


def _input_stage(f_bonds_t, W_i, W_h):
    nblk = N_BONDS // IN_BLK
    bfd = f_bonds_t.shape[0]
    return pl.pallas_call(
        _in_body,
        grid=(nblk,),
        in_specs=[
            pl.BlockSpec((bfd, IN_BLK), lambda i: (0, i)),
            pl.BlockSpec((bfd, HIDDEN), lambda i: (0, 0)),
            pl.BlockSpec((HIDDEN, HIDDEN), lambda i: (0, 0)),
        ],
        out_specs=[
            pl.BlockSpec((IN_BLK, HIDDEN), lambda i: (i, 0)),
            pl.BlockSpec((IN_BLK, HIDDEN), lambda i: (i, 0)),
        ],
        out_shape=[
            jax.ShapeDtypeStruct((N_BONDS, HIDDEN), jnp.float32),
            jax.ShapeDtypeStruct((N_BONDS, HIDDEN), jnp.float32),
        ],
    )(f_bonds_t, W_i, W_h)


def _mm_body(m_ref, wh_ref, h_ref):
    h_ref[...] = jnp.dot(m_ref[...], wh_ref[...],
                         preferred_element_type=jnp.float32)


def _h_stage(m, W_h):
    nblk = N_BONDS // IN_BLK
    return pl.pallas_call(
        _mm_body,
        grid=(nblk,),
        in_specs=[
            pl.BlockSpec((IN_BLK, HIDDEN), lambda i: (i, 0)),
            pl.BlockSpec((HIDDEN, HIDDEN), lambda i: (0, 0)),
        ],
        out_specs=pl.BlockSpec((IN_BLK, HIDDEN), lambda i: (i, 0)),
        out_shape=jax.ShapeDtypeStruct((N_BONDS, HIDDEN), jnp.float32),
    )(m, W_h)


OUT_BLK = 2000


def _out_body(fa_ref, am_ref, woa_ref, wom_ref, bo_ref, out_ref):
    acc = jnp.dot(fa_ref[...], woa_ref[...], preferred_element_type=jnp.float32)
    acc += jnp.dot(am_ref[...], wom_ref[...], preferred_element_type=jnp.float32)
    out_ref[...] = jnp.maximum(acc + bo_ref[...], 0.0)


def _out_stage(f_atoms, am, W_oa, W_om, b_o):
    nblk = N_ATOMS // OUT_BLK
    afd = f_atoms.shape[1]
    return pl.pallas_call(
        _out_body,
        grid=(nblk,),
        in_specs=[
            pl.BlockSpec((OUT_BLK, afd), lambda i: (i, 0)),
            pl.BlockSpec((OUT_BLK, HIDDEN), lambda i: (i, 0)),
            pl.BlockSpec((afd, HIDDEN), lambda i: (0, 0)),
            pl.BlockSpec((HIDDEN, HIDDEN), lambda i: (0, 0)),
            pl.BlockSpec((1, HIDDEN), lambda i: (0, 0)),
        ],
        out_specs=pl.BlockSpec((OUT_BLK, HIDDEN), lambda i: (i, 0)),
        out_shape=jax.ShapeDtypeStruct((N_ATOMS, HIDDEN), jnp.float32),
    )(f_atoms, am, W_oa, W_om, b_o)


def kernel(f_atoms, f_bonds, a2b, b2a, b2revb, W_i, W_h, W_o, b_o):
    pad_idx = (jnp.arange((A_PAD - N_ATOMS) * MAX_NB, dtype=jnp.int32)
               % N_BONDS).reshape(A_PAD - N_ATOMS, MAX_NB)
    a2b_flat = jnp.concatenate(
        [a2b.astype(jnp.int32), pad_idx], axis=0).reshape(-1)
    b2a = b2a.astype(jnp.int32)
    b2revb = b2revb.astype(jnp.int32)

    inp, h = _input_stage(f_bonds.T, W_i, W_h)
    for d in range(DEPTH - 1):
        amh = _seg_sum(h, a2b_flat)
        m = _msg_step(inp, h, amh, b2a, b2revb)
        if d < DEPTH - 2:
            h = _h_stage(m, W_h)
    am = _seg_sum(m, a2b_flat)[:N_ATOMS]
    afd = f_atoms.shape[1]
    return _out_stage(f_atoms, am, W_o[:afd], W_o[afd:],
                      b_o.reshape(1, HIDDEN))


# speedup vs baseline: 3.6703x; 1.0084x over previous
import functools

import jax
import jax.numpy as jnp
from jax import lax
from jax.experimental import pallas as pl
from jax.experimental.pallas import tpu as pltpu
from jax.experimental.pallas import tpu_sc as plsc

N_ATOMS = 10000
N_BONDS = 320000
MAX_NB = 32
HIDDEN = 128
DEPTH = 3

NC = 2
NS = 16
NW = NC * NS

ATOMS_PER_W = 320
A_PAD = NW * ATOMS_PER_W
SEG_ATOMS_PER_CHUNK = 8
SEG_CHUNKS = ATOMS_PER_W // SEG_ATOMS_PER_CHUNK
SEG_NBUF = 3

BONDS_PER_W = N_BONDS // NW
MSG_CHUNK = 80
MSG_CHUNKS = BONDS_PER_W // MSG_CHUNK

_mesh = plsc.VectorSubcoreMesh(core_axis_name="c", subcore_axis_name="s")


def _wid():
    return lax.axis_index("s") * NC + lax.axis_index("c")


def _pipeline(n_chunks, nbuf, issue, wait, compute, store_issue, store_wait):
    def body(c, slot, prefetch, wait_st):
        if prefetch:
            issue(c + nbuf - 1, (slot + nbuf - 1) % nbuf)
        wait(slot)
        if wait_st:
            store_wait(slot)
        compute(c, slot)
        store_issue(c, slot)

    for c in range(nbuf - 1):
        issue(c, c % nbuf)
    tail = nbuf - 1
    tail += (n_chunks - nbuf - tail) % nbuf
    for c in range(min(nbuf, n_chunks - tail)):
        body(c, c % nbuf, True, False)
    groups = (n_chunks - tail - nbuf) // nbuf

    def group(t, carry):
        c0 = nbuf + nbuf * t
        for k in range(nbuf):
            body(c0 + k, k % nbuf, True, True)
        return carry

    lax.fori_loop(0, groups, group, 0)
    for c in range(n_chunks - tail, n_chunks):
        body(c, c % nbuf, c + nbuf - 1 < n_chunks, True)
    for s in range(min(nbuf, n_chunks)):
        store_wait(s)


@functools.partial(
    pl.kernel,
    out_type=jax.ShapeDtypeStruct((A_PAD, HIDDEN), jnp.float32),
    mesh=_mesh,
    scratch_types=[
        pltpu.VMEM((ATOMS_PER_W * MAX_NB,), jnp.int32),
        pltpu.VMEM((SEG_NBUF, SEG_ATOMS_PER_CHUNK * MAX_NB, HIDDEN), jnp.float32),
        pltpu.VMEM((SEG_NBUF, SEG_ATOMS_PER_CHUNK, HIDDEN), jnp.float32),
        pltpu.SemaphoreType.DMA,
        pltpu.SemaphoreType.DMA,
        pltpu.SemaphoreType.DMA,
        pltpu.SemaphoreType.DMA,
        pltpu.SemaphoreType.DMA,
        pltpu.SemaphoreType.DMA,
    ],
)
def _seg_sum(h_hbm, idx_hbm, out_hbm, idx_v, rows_v, acc_v,
             sem_g0, sem_g1, sem_g2, sem_s0, sem_s1, sem_s2):
    wid = _wid()
    sem_g = (sem_g0, sem_g1, sem_g2)
    sem_s = (sem_s0, sem_s1, sem_s2)
    pltpu.sync_copy(
        idx_hbm.at[pl.ds(wid * (ATOMS_PER_W * MAX_NB), ATOMS_PER_W * MAX_NB)],
        idx_v)
    half = SEG_ATOMS_PER_CHUNK * MAX_NB // 2

    def issue(c, slot):
        ib = c * SEG_ATOMS_PER_CHUNK * MAX_NB
        pltpu.async_copy(h_hbm.at[idx_v.at[pl.ds(ib, half)]],
                         rows_v.at[slot].at[pl.ds(0, half)], sem_g[slot])
        pltpu.async_copy(h_hbm.at[idx_v.at[pl.ds(ib + half, half)]],
                         rows_v.at[slot].at[pl.ds(half, half)], sem_g[slot])

    def wait(slot):
        pltpu.make_async_copy(h_hbm.at[pl.ds(0, half)],
                              rows_v.at[slot].at[pl.ds(0, half)],
                              sem_g[slot]).wait()
        pltpu.make_async_copy(h_hbm.at[pl.ds(0, half)],
                              rows_v.at[slot].at[pl.ds(half, half)],
                              sem_g[slot]).wait()

    def compute(c, slot):
        for a in range(SEG_ATOMS_PER_CHUNK):
            def red(j, accs):
                return tuple(
                    accs[g] + rows_v[slot, a * MAX_NB + j, pl.ds(g * 16, 16)]
                    for g in range(8)
                )
            accs = tuple(jnp.zeros((16,), jnp.float32) for _ in range(8))
            accs = lax.fori_loop(0, MAX_NB, red, accs)
            for g in range(8):
                acc_v[slot, a, pl.ds(g * 16, 16)] = accs[g]

    def store_issue(c, slot):
        abase = wid * ATOMS_PER_W + c * SEG_ATOMS_PER_CHUNK
        pltpu.async_copy(acc_v.at[slot],
                         out_hbm.at[pl.ds(abase, SEG_ATOMS_PER_CHUNK)],
                         sem_s[slot])

    def store_wait(slot):
        pltpu.make_async_copy(acc_v.at[slot],
                              out_hbm.at[pl.ds(0, SEG_ATOMS_PER_CHUNK)],
                              sem_s[slot]).wait()

    _pipeline(SEG_CHUNKS, SEG_NBUF, issue, wait, compute, store_issue,
              store_wait)


@functools.partial(
    pl.kernel,
    out_type=jax.ShapeDtypeStruct((N_BONDS, HIDDEN), jnp.float32),
    mesh=_mesh,
    scratch_types=[
        pltpu.VMEM((BONDS_PER_W,), jnp.int32),
        pltpu.VMEM((BONDS_PER_W,), jnp.int32),
        pltpu.VMEM((2, MSG_CHUNK, HIDDEN), jnp.float32),
        pltpu.VMEM((2, MSG_CHUNK, HIDDEN), jnp.float32),
        pltpu.VMEM((2, MSG_CHUNK, HIDDEN), jnp.float32),
        pltpu.VMEM((2, MSG_CHUNK, HIDDEN), jnp.float32),
        pltpu.SemaphoreType.DMA,
        pltpu.SemaphoreType.DMA,
        pltpu.SemaphoreType.DMA,
        pltpu.SemaphoreType.DMA,
    ],
)
def _msg_step(inp_hbm, h_hbm, amh_hbm, b2a_hbm, b2revb_hbm, out_hbm,
              ia_v, ir_v, arow_v, rrow_v, inp_v, out_v,
              sem_g0, sem_g1, sem_s0, sem_s1):
    wid = _wid()
    sem_g = (sem_g0, sem_g1)
    sem_s = (sem_s0, sem_s1)
    pltpu.sync_copy(b2a_hbm.at[pl.ds(wid * BONDS_PER_W, BONDS_PER_W)], ia_v)
    pltpu.sync_copy(b2revb_hbm.at[pl.ds(wid * BONDS_PER_W, BONDS_PER_W)], ir_v)

    def issue(c, slot):
        base = wid * BONDS_PER_W + c * MSG_CHUNK
        pltpu.async_copy(amh_hbm.at[ia_v.at[pl.ds(c * MSG_CHUNK, MSG_CHUNK)]],
                         arow_v.at[slot], sem_g[slot])
        pltpu.async_copy(h_hbm.at[ir_v.at[pl.ds(c * MSG_CHUNK, MSG_CHUNK)]],
                         rrow_v.at[slot], sem_g[slot])
        pltpu.async_copy(inp_hbm.at[pl.ds(base, MSG_CHUNK)],
                         inp_v.at[slot], sem_g[slot])

    def wait(slot):
        pltpu.make_async_copy(amh_hbm.at[pl.ds(0, MSG_CHUNK)],
                              arow_v.at[slot], sem_g[slot]).wait()
        pltpu.make_async_copy(h_hbm.at[pl.ds(0, MSG_CHUNK)],
                              rrow_v.at[slot], sem_g[slot]).wait()
        pltpu.make_async_copy(inp_hbm.at[pl.ds(0, MSG_CHUNK)],
                              inp_v.at[slot], sem_g[slot]).wait()

    def compute(c, slot):
        def row(i, cc):
            for g in range(8):
                sl = pl.ds(g * 16, 16)
                v = inp_v[slot, i, sl] + arow_v[slot, i, sl] - rrow_v[slot, i, sl]
                out_v[slot, i, sl] = jnp.maximum(v, 0.0)
            return cc

        lax.fori_loop(0, MSG_CHUNK, row, 0)

    def store_issue(c, slot):
        base = wid * BONDS_PER_W + c * MSG_CHUNK
        pltpu.async_copy(out_v.at[slot], out_hbm.at[pl.ds(base, MSG_CHUNK)],
                         sem_s[slot])

    def store_wait(slot):
        pltpu.make_async_copy(out_v.at[slot], out_hbm.at[pl.ds(0, MSG_CHUNK)],
                              sem_s[slot]).wait()

    _pipeline(MSG_CHUNKS, 2, issue, wait, compute, store_issue, store_wait)


IN_BLK = 12800


def _in_body(fbt_ref, wi_ref, wh_ref, inp_ref, h_ref):
    inp = jax.lax.dot_general(
        fbt_ref[...], wi_ref[...], (((0,), (0,)), ((), ())),
        preferred_element_type=jnp.float32)
    inp_ref[...] = inp
    h_ref[...] = jnp.dot(jnp.maximum(inp, 0.0), wh_ref[...],
                         preferred_element_type=jnp.float32)


def _input_stage(f_bonds_t, W_i, W_h):
    nblk = N_BONDS // IN_BLK
    bfd = f_bonds_t.shape[0]
    return pl.pallas_call(
        _in_body,
        grid=(nblk,),
        in_specs=[
            pl.BlockSpec((bfd, IN_BLK), lambda i: (0, i)),
            pl.BlockSpec((bfd, HIDDEN), lambda i: (0, 0)),
            pl.BlockSpec((HIDDEN, HIDDEN), lambda i: (0, 0)),
        ],
        out_specs=[
            pl.BlockSpec((IN_BLK, HIDDEN), lambda i: (i, 0)),
            pl.BlockSpec((IN_BLK, HIDDEN), lambda i: (i, 0)),
        ],
        out_shape=[
            jax.ShapeDtypeStruct((N_BONDS, HIDDEN), jnp.float32),
            jax.ShapeDtypeStruct((N_BONDS, HIDDEN), jnp.float32),
        ],
    )(f_bonds_t, W_i, W_h)


def _mm_body(m_ref, wh_ref, h_ref):
    h_ref[...] = jnp.dot(m_ref[...], wh_ref[...],
                         preferred_element_type=jnp.float32)


def _h_stage(m, W_h):
    nblk = N_BONDS // IN_BLK
    return pl.pallas_call(
        _mm_body,
        grid=(nblk,),
        in_specs=[
            pl.BlockSpec((IN_BLK, HIDDEN), lambda i: (i, 0)),
            pl.BlockSpec((HIDDEN, HIDDEN), lambda i: (0, 0)),
        ],
        out_specs=pl.BlockSpec((IN_BLK, HIDDEN), lambda i: (i, 0)),
        out_shape=jax.ShapeDtypeStruct((N_BONDS, HIDDEN), jnp.float32),
    )(m, W_h)


OUT_BLK = 2000


def _out_body(fa_ref, am_ref, woa_ref, wom_ref, bo_ref, out_ref):
    acc = jnp.dot(fa_ref[...], woa_ref[...], preferred_element_type=jnp.float32)
    acc += jnp.dot(am_ref[...], wom_ref[...], preferred_element_type=jnp.float32)
    out_ref[...] = jnp.maximum(acc + bo_ref[...], 0.0)


def _out_stage(f_atoms, am, W_oa, W_om, b_o):
    nblk = N_ATOMS // OUT_BLK
    afd = f_atoms.shape[1]
    return pl.pallas_call(
        _out_body,
        grid=(nblk,),
        in_specs=[
            pl.BlockSpec((OUT_BLK, afd), lambda i: (i, 0)),
            pl.BlockSpec((OUT_BLK, HIDDEN), lambda i: (i, 0)),
            pl.BlockSpec((afd, HIDDEN), lambda i: (0, 0)),
            pl.BlockSpec((HIDDEN, HIDDEN), lambda i: (0, 0)),
            pl.BlockSpec((1, HIDDEN), lambda i: (0, 0)),
        ],
        out_specs=pl.BlockSpec((OUT_BLK, HIDDEN), lambda i: (i, 0)),
        out_shape=jax.ShapeDtypeStruct((N_ATOMS, HIDDEN), jnp.float32),
    )(f_atoms, am, W_oa, W_om, b_o)


def kernel(f_atoms, f_bonds, a2b, b2a, b2revb, W_i, W_h, W_o, b_o):
    pad_idx = (jnp.arange((A_PAD - N_ATOMS) * MAX_NB, dtype=jnp.int32)
               % N_BONDS).reshape(A_PAD - N_ATOMS, MAX_NB)
    a2b_flat = jnp.concatenate(
        [a2b.astype(jnp.int32), pad_idx], axis=0).reshape(-1)
    b2a = b2a.astype(jnp.int32)
    b2revb = b2revb.astype(jnp.int32)

    inp, h = _input_stage(f_bonds.T, W_i, W_h)
    for d in range(DEPTH - 1):
        amh = _seg_sum(h, a2b_flat)
        m = _msg_step(inp, h, amh, b2a, b2revb)
        if d < DEPTH - 2:
            h = _h_stage(m, W_h)
    am = _seg_sum(m, a2b_flat)[:N_ATOMS]
    afd = f_atoms.shape[1]
    return _out_stage(f_atoms, am, W_o[:afd], W_o[afd:],
                      b_o.reshape(1, HIDDEN))
